# Initial kernel scaffold; baseline (speedup 1.0000x reference)
import jax
import jax.numpy as jnp
from jax.experimental import pallas as pl


def kernel(inputs, token_table):
    raise NotImplementedError("write your pallas kernel here")


# speedup vs baseline: 3.4685x; 3.4685x over previous
import functools

import jax
import jax.numpy as jnp
import numpy as np
from jax import lax
from jax.experimental import pallas as pl
from jax.experimental.pallas import tpu as pltpu
from jax.experimental.pallas import tpu_sc as plsc

SEQ = 200
DIM = 64
BATCH = 4096
ROWS = BATCH * SEQ
NC, NS, LANES = 2, 16, 16
NW = NC * NS
SEQ_PER_W = BATCH // NW
ROWS_PER_W = SEQ_PER_W * SEQ
CHUNK_SEQS = 2
CHUNK = CHUNK_SEQS * SEQ
N_CHUNKS = SEQ_PER_W // CHUNK_SEQS
IDX_GRP = 100
GRPS = CHUNK // IDX_GRP


def _pos_encoding():
    half = DIM // 2
    positions = np.arange(SEQ).reshape(SEQ, 1)
    depths = np.arange(half).reshape(1, half) / half
    angle_rates = 1 / 10000 ** depths
    angle_rads = positions * angle_rates
    return np.concatenate([np.sin(angle_rads), np.cos(angle_rads)], axis=-1).astype(np.float32)


def _body(idx_hbm, table_hbm, pos_hbm, out_hbm, idx_v, rows_v, pos_v, sem):
    wid = lax.axis_index("s") * NC + lax.axis_index("c")
    pltpu.sync_copy(pos_hbm, pos_v)

    def chunk_body(c, carry):
        base = wid * ROWS_PER_W + c * CHUNK
        g0 = wid * (ROWS_PER_W // IDX_GRP) + c * GRPS
        pltpu.sync_copy(idx_hbm.at[pl.ds(g0, GRPS)], idx_v)
        descs = [
            pltpu.async_copy(
                table_hbm.at[idx_v.at[g]],
                rows_v.at[pl.ds(g * IDX_GRP, IDX_GRP)],
                sem,
            )
            for g in range(GRPS)
        ]
        for d in descs:
            d.wait()

        def pos_body(p, carry2):
            pv = [pos_v[p, pl.ds(i * LANES, LANES)] for i in range(DIM // LANES)]
            for s in range(CHUNK_SEQS):
                for i in range(DIM // LANES):
                    plsc.addupdate(rows_v.at[s * SEQ + p, pl.ds(i * LANES, LANES)], pv[i])
            return carry2

        lax.fori_loop(0, SEQ, pos_body, 0)
        pltpu.sync_copy(rows_v, out_hbm.at[pl.ds(base, CHUNK)])
        return carry

    lax.fori_loop(0, N_CHUNKS, chunk_body, 0)


@functools.partial(jax.jit, static_argnums=())
def _run(idx, table, pos):
    kern = pl.kernel(
        _body,
        out_type=jax.ShapeDtypeStruct((ROWS, DIM), jnp.float32),
        mesh=plsc.VectorSubcoreMesh(core_axis_name="c", subcore_axis_name="s"),
        scratch_types=[
            pltpu.VMEM((GRPS, IDX_GRP), jnp.int32),
            pltpu.VMEM((CHUNK, DIM), jnp.float32),
            pltpu.VMEM((SEQ, DIM), jnp.float32),
            pltpu.SemaphoreType.DMA,
        ],
        compiler_params=pltpu.CompilerParams(use_tc_tiling_on_sc=False),
    )
    return kern(idx, table, pos)


def kernel(inputs, token_table):
    idx = inputs.astype(jnp.int32).reshape(ROWS // IDX_GRP, IDX_GRP)
    pos = jnp.asarray(_pos_encoding())
    out = _run(idx, token_table, pos)
    return out.reshape(BATCH, SEQ, DIM)


# speedup vs baseline: 4.2323x; 1.2202x over previous
import functools

import jax
import jax.numpy as jnp
import numpy as np
from jax import lax
from jax.experimental import pallas as pl
from jax.experimental.pallas import tpu as pltpu
from jax.experimental.pallas import tpu_sc as plsc

SEQ = 200
DIM = 64
BATCH = 4096
ROWS = BATCH * SEQ
NC, NS, LANES = 2, 16, 16
NW = NC * NS
SEQ_PER_W = BATCH // NW
ROWS_PER_W = SEQ_PER_W * SEQ
CHUNK = SEQ
N_CHUNKS = SEQ_PER_W
IDX_GRP = 100
GRPS = CHUNK // IDX_GRP
GRP_PER_W = ROWS_PER_W // IDX_GRP
NBUF = 4
LOOKAHEAD = 2


def _pos_encoding():
    half = DIM // 2
    positions = np.arange(SEQ).reshape(SEQ, 1)
    depths = np.arange(half).reshape(1, half) / half
    angle_rates = 1 / 10000 ** depths
    angle_rads = positions * angle_rates
    return np.concatenate([np.sin(angle_rads), np.cos(angle_rads)], axis=-1).astype(np.float32)


def _body(idx_hbm, table_hbm, pos_hbm, out_hbm,
          idx_all, r0, r1, r2, r3, pos_v,
          si0, si1, si2, si3, so0, so1, so2, so3):
    rows = (r0, r1, r2, r3)
    sin = (si0, si1, si2, si3)
    sout = (so0, so1, so2, so3)
    wid = lax.axis_index("s") * NC + lax.axis_index("c")
    base = wid * ROWS_PER_W

    pltpu.sync_copy(pos_hbm, pos_v)
    pltpu.sync_copy(idx_hbm.at[pl.ds(wid * GRP_PER_W, GRP_PER_W)], idx_all)

    def fire_gather(c, b):
        for g in range(GRPS):
            pltpu.async_copy(
                table_hbm.at[idx_all.at[c * GRPS + g]],
                rows[b].at[pl.ds(g * IDX_GRP, IDX_GRP)],
                sin[b],
            )

    def wait_in(b):
        pltpu.make_async_copy(table_hbm.at[pl.ds(0, CHUNK)], rows[b], sin[b]).wait()

    def fire_out(c, b):
        pltpu.async_copy(rows[b], out_hbm.at[pl.ds(base + c * CHUNK, CHUNK)], sout[b])

    def wait_out(b):
        pltpu.make_async_copy(rows[b], out_hbm.at[pl.ds(0, CHUNK)], sout[b]).wait()

    def compute(b):
        def row_body(r, carry):
            for i in range(DIM // LANES):
                plsc.addupdate(
                    rows[b].at[r, pl.ds(i * LANES, LANES)],
                    pos_v[r, pl.ds(i * LANES, LANES)],
                )
            return carry

        lax.fori_loop(0, SEQ, row_body, 0)

    fire_gather(0, 0)
    fire_gather(1, 1)

    for c in (0, 1):
        fire_gather(c + LOOKAHEAD, c + LOOKAHEAD)
        wait_in(c)
        compute(c)
        fire_out(c, c)

    def outer(t, carry):
        for j in range(NBUF):
            c = 2 + t * NBUF + j
            wait_out(j)
            fire_gather(c + LOOKAHEAD, j)
            b = (j + LOOKAHEAD) % NBUF
            wait_in(b)
            compute(b)
            fire_out(c, b)
        return carry

    lax.fori_loop(0, (N_CHUNKS - NBUF) // NBUF, outer, 0)

    for c in (N_CHUNKS - 2, N_CHUNKS - 1):
        b = c % NBUF
        wait_in(b)
        compute(b)
        fire_out(c, b)

    for b in range(NBUF):
        wait_out(b)


@functools.partial(jax.jit, static_argnums=())
def _run(idx, table, pos):
    kern = pl.kernel(
        _body,
        out_type=jax.ShapeDtypeStruct((ROWS, DIM), jnp.float32),
        mesh=plsc.VectorSubcoreMesh(core_axis_name="c", subcore_axis_name="s"),
        scratch_types=[
            pltpu.VMEM((GRP_PER_W, IDX_GRP), jnp.int32),
            pltpu.VMEM((CHUNK, DIM), jnp.float32),
            pltpu.VMEM((CHUNK, DIM), jnp.float32),
            pltpu.VMEM((CHUNK, DIM), jnp.float32),
            pltpu.VMEM((CHUNK, DIM), jnp.float32),
            pltpu.VMEM((SEQ, DIM), jnp.float32),
            pltpu.SemaphoreType.DMA,
            pltpu.SemaphoreType.DMA,
            pltpu.SemaphoreType.DMA,
            pltpu.SemaphoreType.DMA,
            pltpu.SemaphoreType.DMA,
            pltpu.SemaphoreType.DMA,
            pltpu.SemaphoreType.DMA,
            pltpu.SemaphoreType.DMA,
        ],
        compiler_params=pltpu.CompilerParams(use_tc_tiling_on_sc=False),
    )
    return kern(idx, table, pos)


def kernel(inputs, token_table):
    idx = inputs.astype(jnp.int32).reshape(ROWS // IDX_GRP, IDX_GRP)
    pos = jnp.asarray(_pos_encoding())
    out = _run(idx, token_table, pos)
    return out.reshape(BATCH, SEQ, DIM)
